# baseline (speedup 1.0000x reference)
import functools

import jax
import jax.numpy as jnp
from jax import lax
from jax.experimental import pallas as pl
from jax.experimental.pallas import tpu as pltpu
from jax.experimental.pallas import tpu_sc as plsc

NW = 32
LANES = 16
NBUF = 4
SPG = 128
PD = 128


def _emb_mean_kernel(B, L, D, idx_hbm, table_hbm, out_hbm,
                     idx_v, rows0, rows1, rows2, rows3, acc_v,
                     sem0, sem1, sem2, sem3):
    BPW = B // NW
    NBLK = BPW // SPG
    NV = D // LANES
    NS = L * NBLK
    inv = jnp.float32(1.0 / L)

    nc = plsc.get_sparse_core_info().num_cores
    wid = lax.axis_index("s") * nc + lax.axis_index("c")

    pltpu.sync_copy(idx_hbm.at[:, pl.ds(wid * BPW, BPW)], idx_v)

    def zbody(r, carry):
        for k in range(PD // LANES):
            acc_v[r, pl.ds(k * LANES, LANES)] = jnp.zeros((LANES,), jnp.float32)
        return carry

    lax.fori_loop(0, BPW // 2, zbody, 0)

    bufs = (rows0, rows1, rows2, rows3)
    sems = (sem0, sem1, sem2, sem3)

    def start(s, b):
        l = s // NBLK
        blk = s - l * NBLK
        idx_slice = idx_v.at[l, pl.ds(blk * SPG, SPG)]
        pltpu.async_copy(table_hbm.at[idx_slice], bufs[b], sems[b])

    def wait(b):
        pltpu.make_async_copy(
            table_hbm.at[idx_v.at[0, pl.ds(0, SPG)]], bufs[b], sems[b]
        ).wait()

    def accumulate(s, b):
        blk = s - (s // NBLK) * NBLK
        base = blk * SPG
        rows = bufs[b]

        def abody(r, carry):
            row = base + r
            q = row // 2
            off = (row % 2) * D
            for k in range(NV):
                plsc.addupdate(
                    acc_v.at[q, pl.ds(off + k * LANES, LANES)],
                    rows[r, pl.ds(k * LANES, LANES)],
                )
            return carry

        lax.fori_loop(0, SPG, abody, 0)

    for b in range(NBUF):
        start(b, b)

    def body(g, carry):
        for b in range(NBUF):
            s = NBUF * g + b
            wait(b)
            accumulate(s, b)

            @pl.when(s + NBUF < NS)
            def _():
                start(s + NBUF, b)
        return carry

    lax.fori_loop(0, NS // NBUF, body, 0)

    def sbody(r, carry):
        for k in range(PD // LANES):
            sl = pl.ds(k * LANES, LANES)
            acc_v[r, sl] = acc_v[r, sl] * inv
        return carry

    lax.fori_loop(0, BPW // 2, sbody, 0)
    pltpu.sync_copy(acc_v, out_hbm.at[pl.ds(wid * (BPW // 2), BPW // 2)])


@functools.partial(jax.jit, static_argnames=("B", "L", "D"))
def _emb_mean(idx_t, W128, B, L, D):
    BPW = B // NW
    mesh = plsc.VectorSubcoreMesh(core_axis_name="c", subcore_axis_name="s")
    out = pl.kernel(
        functools.partial(_emb_mean_kernel, B, L, D),
        out_type=jax.ShapeDtypeStruct((B // 2, PD), jnp.float32),
        mesh=mesh,
        compiler_params=pltpu.CompilerParams(use_tc_tiling_on_sc=True),
        scratch_types=[
            pltpu.VMEM((L, BPW), jnp.int32),
            pltpu.VMEM((SPG, PD), jnp.float32),
            pltpu.VMEM((SPG, PD), jnp.float32),
            pltpu.VMEM((SPG, PD), jnp.float32),
            pltpu.VMEM((SPG, PD), jnp.float32),
            pltpu.VMEM((BPW // 2, PD), jnp.float32),
            pltpu.SemaphoreType.DMA,
            pltpu.SemaphoreType.DMA,
            pltpu.SemaphoreType.DMA,
            pltpu.SemaphoreType.DMA,
        ],
    )(idx_t, W128)
    return out.reshape(B, D)


def kernel(word_ids, W):
    B, L = word_ids.shape
    D = W.shape[1]
    BPW = B // NW
    assert B % NW == 0 and BPW % SPG == 0 and D % LANES == 0
    idx_t = word_ids.astype(jnp.int32).T
    W128 = jnp.pad(W, ((0, 0), (0, PD - D)))
    return _emb_mean(idx_t, W128, B, L, D)
